# Initial kernel scaffold; baseline (speedup 1.0000x reference)
import jax
import jax.numpy as jnp
from jax.experimental import pallas as pl


def kernel(x, edge_index, W_pre, b_pre, Wl1, Wr1, b1, Wl2, Wr2, b2):
    raise NotImplementedError("write your pallas kernel here")


# speedup vs baseline: 2.9456x; 2.9456x over previous
import functools
import jax
import jax.numpy as jnp
from jax import lax
from jax.experimental import pallas as pl
from jax.experimental.pallas import tpu as pltpu
from jax.experimental.pallas import tpu_sc as plsc

N_NODES = 10000
N_EDGES = 160000
D = 256
DH = 128
NSUB = 16
NP = 10240
STRIPE = NP // NSUB
EPS = N_EDGES // NSUB
B = 80
NB = EPS // B
EPC = N_EDGES // 32
BC = 40
NBC = EPC // BC
MB = 1000


def _sc_mesh():
  return plsc.VectorSubcoreMesh(core_axis_name="c", subcore_axis_name="s")


def _counts(dst, zeros_c, ones_h):
  @functools.partial(
      pl.kernel,
      mesh=_sc_mesh(),
      out_type=jax.ShapeDtypeStruct((2 * NP, DH), jnp.float32),
      scratch_types=[
          pltpu.VMEM((BC,), jnp.int32),
          pltpu.VMEM((BC, DH), jnp.float32),
          pltpu.VMEM_SHARED((NP, DH), jnp.float32),
      ],
  )
  def k(dst_hbm, zc_hbm, ones_hbm, cnt_hbm, idx_d, ones_v, cacc):
    cid = lax.axis_index("c")
    sid = lax.axis_index("s")
    r0 = sid * STRIPE
    pltpu.sync_copy(zc_hbm.at[pl.ds(r0, STRIPE)], cacc.at[pl.ds(r0, STRIPE)])
    pltpu.sync_copy(ones_hbm, ones_v)
    plsc.subcore_barrier()

    e0 = (cid * NSUB + sid) * EPC

    def body(g, carry):
      pltpu.sync_copy(dst_hbm.at[pl.ds(e0 + g * BC, BC)], idx_d)
      pltpu.sync_copy(ones_v, cacc.at[idx_d], add=True)
      return carry

    lax.fori_loop(0, NBC, body, 0)
    plsc.subcore_barrier()
    pltpu.sync_copy(cacc.at[pl.ds(r0, STRIPE)],
                    cnt_hbm.at[pl.ds(cid * NP + r0, STRIPE)])

  return k(dst, zeros_c, ones_h)


def _agg(table, src2, dst, zeros_h):
  @functools.partial(
      pl.kernel,
      mesh=_sc_mesh(),
      out_type=jax.ShapeDtypeStruct((2 * NP, DH), jnp.float32),
      scratch_types=[
          pltpu.VMEM((B,), jnp.int32),
          pltpu.VMEM((B,), jnp.int32),
          pltpu.VMEM((B, DH), jnp.float32),
          pltpu.VMEM_SHARED((NP, DH), jnp.float32),
          pltpu.SemaphoreType.DMA,
      ],
  )
  def k(table_hbm, src2_hbm, dst_hbm, zh_hbm, sum_hbm,
        idx_s, idx_d, rows, acc, sem):
    cid = lax.axis_index("c")
    sid = lax.axis_index("s")
    r0 = sid * STRIPE
    pltpu.sync_copy(zh_hbm.at[pl.ds(r0, STRIPE)], acc.at[pl.ds(r0, STRIPE)])
    plsc.subcore_barrier()

    e0 = cid * N_EDGES + sid * EPS

    def body(g, carry):
      base = e0 + g * B
      pltpu.sync_copy(src2_hbm.at[pl.ds(base, B)], idx_s)
      pltpu.sync_copy(dst_hbm.at[pl.ds(sid * EPS + g * B, B)], idx_d)
      pltpu.async_copy(table_hbm.at[idx_s], rows, sem).wait()
      pltpu.sync_copy(rows, acc.at[idx_d], add=True)
      return carry

    lax.fori_loop(0, NB, body, 0)
    plsc.subcore_barrier()
    pltpu.sync_copy(acc.at[pl.ds(r0, STRIPE)],
                    sum_hbm.at[pl.ds(cid * NP + r0, STRIPE)])

  return k(table, src2, dst, zeros_h)


def _tc_linear(x, W, b2):
  def body(x_ref, w_ref, b_ref, o_ref):
    o_ref[...] = lax.dot_general(
        x_ref[...], w_ref[...], (((1,), (1,)), ((), ())),
        preferred_element_type=jnp.float32) + b_ref[...]

  return pl.pallas_call(
      body,
      grid=(N_NODES // MB,),
      in_specs=[
          pl.BlockSpec((MB, D), lambda i: (i, 0)),
          pl.BlockSpec((D, D), lambda i: (0, 0)),
          pl.BlockSpec((1, D), lambda i: (0, 0)),
      ],
      out_specs=pl.BlockSpec((MB, D), lambda i: (i, 0)),
      out_shape=jax.ShapeDtypeStruct((N_NODES, D), jnp.float32),
  )(x, W, b2)


def _tc_combine(sumL, sumR, cntA, cntB, h, WlA, WlB, Wr, b2, relu):
  def body(sl_ref, sr_ref, ca_ref, cb_ref, h_ref, wa_ref, wb_ref, wr_ref,
           b_ref, o_ref):
    cnt = ca_ref[:, 0:1] + cb_ref[:, 0:1]
    inv = 1.0 / jnp.maximum(cnt, 1.0)
    acc = lax.dot_general(sl_ref[...] * inv, wa_ref[...],
                          (((1,), (1,)), ((), ())),
                          preferred_element_type=jnp.float32)
    acc = acc + lax.dot_general(sr_ref[...] * inv, wb_ref[...],
                                (((1,), (1,)), ((), ())),
                                preferred_element_type=jnp.float32)
    acc = acc + lax.dot_general(h_ref[...], wr_ref[...],
                                (((1,), (1,)), ((), ())),
                                preferred_element_type=jnp.float32)
    acc = acc + b_ref[...]
    if relu:
      acc = jnp.maximum(acc, 0.0)
    o_ref[...] = acc

  return pl.pallas_call(
      body,
      grid=(N_NODES // MB,),
      in_specs=[
          pl.BlockSpec((MB, DH), lambda i: (i, 0)),
          pl.BlockSpec((MB, DH), lambda i: (i, 0)),
          pl.BlockSpec((MB, 16), lambda i: (i, 0)),
          pl.BlockSpec((MB, 16), lambda i: (i, 0)),
          pl.BlockSpec((MB, D), lambda i: (i, 0)),
          pl.BlockSpec((D, DH), lambda i: (0, 0)),
          pl.BlockSpec((D, DH), lambda i: (0, 0)),
          pl.BlockSpec((D, D), lambda i: (0, 0)),
          pl.BlockSpec((1, D), lambda i: (0, 0)),
      ],
      out_specs=pl.BlockSpec((MB, D), lambda i: (i, 0)),
      out_shape=jax.ShapeDtypeStruct((N_NODES, D), jnp.float32),
  )(sumL, sumR, cntA, cntB, h, WlA, WlB, Wr, b2)


def kernel(x, edge_index, W_pre, b_pre, Wl1, Wr1, b1, Wl2, Wr2, b2):
  src = edge_index[0].astype(jnp.int32)
  dst = edge_index[1].astype(jnp.int32)
  src2 = jnp.concatenate([src * 2, src * 2 + 1])
  zeros_h = jnp.zeros((NP, DH), jnp.float32)
  zeros_c = jnp.zeros((NP, DH), jnp.float32)
  ones_h = jnp.ones((BC, DH), jnp.float32)
  bp2 = b_pre.reshape(1, D)
  b12 = b1.reshape(1, D)
  b22 = b2.reshape(1, D)

  cnt2 = _counts(dst, zeros_c, ones_h)
  cntA, cntB = cnt2[:NP, :16], cnt2[NP:, :16]
  h0 = _tc_linear(x, W_pre, bp2)
  sums1 = _agg(h0.reshape(2 * N_NODES, DH), src2, dst, zeros_h)
  h1 = _tc_combine(sums1[:NP], sums1[NP:], cntA, cntB, h0,
                   Wl1[:, :DH], Wl1[:, DH:], Wr1, b12, True)
  sums2 = _agg(h1.reshape(2 * N_NODES, DH), src2, dst, zeros_h)
  out = _tc_combine(sums2[:NP], sums2[NP:], cntA, cntB, h1,
                    Wl2[:, :DH], Wl2[:, DH:], Wr2, b22, False)
  return out


# speedup vs baseline: 6.3858x; 2.1679x over previous
import functools
import jax
import jax.numpy as jnp
from jax import lax
from jax.experimental import pallas as pl
from jax.experimental.pallas import tpu as pltpu
from jax.experimental.pallas import tpu_sc as plsc

N_NODES = 10000
N_EDGES = 160000
D = 256
DH = 128
NSUB = 16
NP = 10240
STRIPE = NP // NSUB
EPS = N_EDGES // NSUB
B = 128
EPSP = 10240
NB = EPSP // B
EPC = N_EDGES // 32
BC = 40
NBC = EPC // BC
MB = 1000


def _sc_mesh():
  return plsc.VectorSubcoreMesh(core_axis_name="c", subcore_axis_name="s")


def _counts(dst3, zeros_c, ones_h):
  @functools.partial(
      pl.kernel,
      mesh=_sc_mesh(),
      out_type=jax.ShapeDtypeStruct((2 * NP, DH), jnp.float32),
      scratch_types=[
          pltpu.VMEM((NBC, BC), jnp.int32),
          pltpu.VMEM((BC, DH), jnp.float32),
          pltpu.VMEM_SHARED((NP, DH), jnp.float32),
      ],
  )
  def k(dst_hbm, zc_hbm, ones_hbm, cnt_hbm, didx, ones_v, cacc):
    cid = lax.axis_index("c")
    sid = lax.axis_index("s")
    w = cid * NSUB + sid
    r0 = sid * STRIPE
    pltpu.sync_copy(zc_hbm.at[pl.ds(r0, STRIPE)], cacc.at[pl.ds(r0, STRIPE)])
    pltpu.sync_copy(ones_hbm, ones_v)
    pltpu.sync_copy(dst_hbm.at[w], didx)
    plsc.subcore_barrier()

    def body(g, carry):
      pltpu.sync_copy(ones_v, cacc.at[didx.at[g]], add=True)
      return carry

    lax.fori_loop(0, NBC, body, 0)
    plsc.subcore_barrier()
    pltpu.sync_copy(cacc.at[pl.ds(r0, STRIPE)],
                    cnt_hbm.at[pl.ds(cid * NP + r0, STRIPE)])

  return k(dst3, zeros_c, ones_h)


def _agg(table, src3, dst3, zeros_h):
  @functools.partial(
      pl.kernel,
      mesh=_sc_mesh(),
      out_type=jax.ShapeDtypeStruct((2 * NP, DH), jnp.float32),
      scratch_types=[
          pltpu.VMEM((NB + 2, B), jnp.int32),
          pltpu.VMEM((2, B), jnp.int32),
          pltpu.VMEM((B, DH), jnp.float32),
          pltpu.VMEM((B, DH), jnp.float32),
          pltpu.VMEM_SHARED((NP, DH), jnp.float32),
          pltpu.SemaphoreType.DMA,
          pltpu.SemaphoreType.DMA,
          pltpu.SemaphoreType.DMA,
          pltpu.SemaphoreType.DMA,
      ],
  )
  def k(table_hbm, src_hbm, dst_hbm, zh_hbm, sum_hbm,
        gidx, didx, rows0, rows1, acc, sem0, sem1, semd0, semd1):
    cid = lax.axis_index("c")
    sid = lax.axis_index("s")
    w = cid * NSUB + sid
    r0 = sid * STRIPE
    pltpu.sync_copy(src_hbm.at[w], gidx)
    pltpu.sync_copy(zh_hbm.at[pl.ds(r0, STRIPE)], acc.at[pl.ds(r0, STRIPE)])
    plsc.subcore_barrier()

    pltpu.async_copy(table_hbm.at[gidx.at[0]], rows0, sem0)
    pltpu.async_copy(table_hbm.at[gidx.at[1]], rows1, sem1)
    pltpu.async_copy(dst_hbm.at[sid, 0], didx.at[0], semd0)
    pltpu.async_copy(dst_hbm.at[sid, 1], didx.at[1], semd1)

    def body(s, carry):
      g = 2 * s
      pltpu.make_async_copy(table_hbm.at[gidx.at[0]], rows0, sem0).wait()
      pltpu.make_async_copy(dst_hbm.at[sid, 0], didx.at[0], semd0).wait()
      pltpu.sync_copy(rows0, acc.at[didx.at[0]], add=True)
      pltpu.async_copy(table_hbm.at[gidx.at[g + 2]], rows0, sem0)
      pltpu.async_copy(dst_hbm.at[sid, g + 2], didx.at[0], semd0)
      pltpu.make_async_copy(table_hbm.at[gidx.at[1]], rows1, sem1).wait()
      pltpu.make_async_copy(dst_hbm.at[sid, 1], didx.at[1], semd1).wait()
      pltpu.sync_copy(rows1, acc.at[didx.at[1]], add=True)
      pltpu.async_copy(table_hbm.at[gidx.at[g + 3]], rows1, sem1)
      pltpu.async_copy(dst_hbm.at[sid, g + 3], didx.at[1], semd1)
      return carry

    lax.fori_loop(0, NB // 2, body, 0)
    pltpu.make_async_copy(table_hbm.at[gidx.at[0]], rows0, sem0).wait()
    pltpu.make_async_copy(table_hbm.at[gidx.at[1]], rows1, sem1).wait()
    pltpu.make_async_copy(dst_hbm.at[sid, 0], didx.at[0], semd0).wait()
    pltpu.make_async_copy(dst_hbm.at[sid, 1], didx.at[1], semd1).wait()

    plsc.subcore_barrier()
    pltpu.sync_copy(acc.at[pl.ds(r0, STRIPE)],
                    sum_hbm.at[pl.ds(cid * NP + r0, STRIPE)])

  return k(table, src3, dst3, zeros_h)


def _tc_linear(x, W, b2):
  def body(x_ref, w_ref, b_ref, o_ref):
    o_ref[...] = lax.dot_general(
        x_ref[...], w_ref[...], (((1,), (1,)), ((), ())),
        preferred_element_type=jnp.float32) + b_ref[...]

  return pl.pallas_call(
      body,
      grid=(N_NODES // MB,),
      in_specs=[
          pl.BlockSpec((MB, D), lambda i: (i, 0)),
          pl.BlockSpec((D, D), lambda i: (0, 0)),
          pl.BlockSpec((1, D), lambda i: (0, 0)),
      ],
      out_specs=pl.BlockSpec((MB, D), lambda i: (i, 0)),
      out_shape=jax.ShapeDtypeStruct((N_NODES, D), jnp.float32),
  )(x, W, b2)


def _tc_combine(sumL, sumR, cntA, cntB, h, WlA, WlB, Wr, b2, relu):
  def body(sl_ref, sr_ref, ca_ref, cb_ref, h_ref, wa_ref, wb_ref, wr_ref,
           b_ref, o_ref):
    cnt = ca_ref[:, 0:1] + cb_ref[:, 0:1]
    inv = 1.0 / jnp.maximum(cnt, 1.0)
    acc = lax.dot_general(sl_ref[...] * inv, wa_ref[...],
                          (((1,), (1,)), ((), ())),
                          preferred_element_type=jnp.float32)
    acc = acc + lax.dot_general(sr_ref[...] * inv, wb_ref[...],
                                (((1,), (1,)), ((), ())),
                                preferred_element_type=jnp.float32)
    acc = acc + lax.dot_general(h_ref[...], wr_ref[...],
                                (((1,), (1,)), ((), ())),
                                preferred_element_type=jnp.float32)
    acc = acc + b_ref[...]
    if relu:
      acc = jnp.maximum(acc, 0.0)
    o_ref[...] = acc

  return pl.pallas_call(
      body,
      grid=(N_NODES // MB,),
      in_specs=[
          pl.BlockSpec((MB, DH), lambda i: (i, 0)),
          pl.BlockSpec((MB, DH), lambda i: (i, 0)),
          pl.BlockSpec((MB, 16), lambda i: (i, 0)),
          pl.BlockSpec((MB, 16), lambda i: (i, 0)),
          pl.BlockSpec((MB, D), lambda i: (i, 0)),
          pl.BlockSpec((D, DH), lambda i: (0, 0)),
          pl.BlockSpec((D, DH), lambda i: (0, 0)),
          pl.BlockSpec((D, D), lambda i: (0, 0)),
          pl.BlockSpec((1, D), lambda i: (0, 0)),
      ],
      out_specs=pl.BlockSpec((MB, D), lambda i: (i, 0)),
      out_shape=jax.ShapeDtypeStruct((N_NODES, D), jnp.float32),
  )(sumL, sumR, cntA, cntB, h, WlA, WlB, Wr, b2)


def kernel(x, edge_index, W_pre, b_pre, Wl1, Wr1, b1, Wl2, Wr2, b2):
  src = edge_index[0].astype(jnp.int32)
  dst = edge_index[1].astype(jnp.int32)
  npad = EPSP + 2 * B - EPS
  src2 = jnp.concatenate([src * 2, src * 2 + 1]).reshape(32, EPS)
  padg = ((jnp.arange(32, dtype=jnp.int32)[:, None] * 499
           + jnp.arange(npad, dtype=jnp.int32)[None, :]) % (2 * N_NODES))
  src3 = jnp.concatenate([src2, padg], axis=1).reshape(32, NB + 2, B)
  padd = (N_NODES
          + (jnp.arange(npad, dtype=jnp.int32) % (NP - N_NODES)))
  dst3 = jnp.concatenate(
      [dst.reshape(NSUB, EPS),
       jnp.broadcast_to(padd, (NSUB, npad))], axis=1).reshape(NSUB, NB + 2, B)
  dstc = dst.reshape(32, NBC, BC)
  zeros_h = jnp.zeros((NP, DH), jnp.float32)
  zeros_c = jnp.zeros((NP, DH), jnp.float32)
  ones_h = jnp.ones((BC, DH), jnp.float32)
  bp2 = b_pre.reshape(1, D)
  b12 = b1.reshape(1, D)
  b22 = b2.reshape(1, D)

  cnt2 = _counts(dstc, zeros_c, ones_h)
  cntA, cntB = cnt2[:NP, :16], cnt2[NP:, :16]
  h0 = _tc_linear(x, W_pre, bp2)
  sums1 = _agg(h0.reshape(2 * N_NODES, DH), src3, dst3, zeros_h)
  h1 = _tc_combine(sums1[:NP], sums1[NP:], cntA, cntB, h0,
                   Wl1[:, :DH], Wl1[:, DH:], Wr1, b12, True)
  sums2 = _agg(h1.reshape(2 * N_NODES, DH), src3, dst3, zeros_h)
  out = _tc_combine(sums2[:NP], sums2[NP:], cntA, cntB, h1,
                    Wl2[:, :DH], Wl2[:, DH:], Wr2, b22, False)
  return out


# speedup vs baseline: 7.0641x; 1.1062x over previous
import functools
import jax
import jax.numpy as jnp
from jax import lax
from jax.experimental import pallas as pl
from jax.experimental.pallas import tpu as pltpu
from jax.experimental.pallas import tpu_sc as plsc

N_NODES = 10000
N_EDGES = 160000
D = 256
DH = 128
NSUB = 16
NP = 10240
STRIPE = NP // NSUB
EPS = N_EDGES // NSUB
B = 128
EPSP = 10240
NB = EPSP // B
EPC = N_EDGES // 32
BC = 40
NBC = EPC // BC
MB = 1000
CW = 16


def _sc_mesh():
  return plsc.VectorSubcoreMesh(core_axis_name="c", subcore_axis_name="s")


def _counts(dst3c, zeros_h, ones_h):
  @functools.partial(
      pl.kernel,
      mesh=_sc_mesh(),
      out_type=[
          jax.ShapeDtypeStruct((NP, DH), jnp.float32),
          jax.ShapeDtypeStruct((NP, DH), jnp.float32),
      ],
      scratch_types=[
          pltpu.VMEM((NBC, BC), jnp.int32),
          pltpu.VMEM((BC, DH), jnp.float32),
          pltpu.VMEM_SHARED((NP, DH), jnp.float32),
      ],
  )
  def k(dst_hbm, zh_hbm, ones_hbm, cntA_hbm, cntB_hbm, didx, ones_v, cacc):
    cid = lax.axis_index("c")
    sid = lax.axis_index("s")
    w = cid * NSUB + sid
    r0 = sid * STRIPE
    pltpu.sync_copy(zh_hbm.at[pl.ds(r0, STRIPE)], cacc.at[pl.ds(r0, STRIPE)])
    pltpu.sync_copy(ones_hbm, ones_v)
    pltpu.sync_copy(dst_hbm.at[w], didx)
    plsc.subcore_barrier()

    def body(g, carry):
      pltpu.sync_copy(ones_v, cacc.at[didx.at[g]], add=True)
      return carry

    lax.fori_loop(0, NBC, body, 0)
    plsc.subcore_barrier()

    @pl.when(cid == 0)
    def _():
      pltpu.sync_copy(cacc.at[pl.ds(r0, STRIPE)],
                      cntA_hbm.at[pl.ds(r0, STRIPE)])

    @pl.when(cid == 1)
    def _():
      pltpu.sync_copy(cacc.at[pl.ds(r0, STRIPE)],
                      cntB_hbm.at[pl.ds(r0, STRIPE)])

  return k(dst3c, zeros_h, ones_h)


def _agg(tabL, tabR, src3, dst3, zeros_h):
  @functools.partial(
      pl.kernel,
      mesh=_sc_mesh(),
      out_type=[
          jax.ShapeDtypeStruct((NP, DH), jnp.float32),
          jax.ShapeDtypeStruct((NP, DH), jnp.float32),
      ],
      scratch_types=[
          pltpu.VMEM((NB + 2, B), jnp.int32),
          pltpu.VMEM((2, B), jnp.int32),
          pltpu.VMEM((B, DH), jnp.float32),
          pltpu.VMEM((B, DH), jnp.float32),
          pltpu.VMEM_SHARED((NP, DH), jnp.float32),
          pltpu.SemaphoreType.DMA,
          pltpu.SemaphoreType.DMA,
          pltpu.SemaphoreType.DMA,
          pltpu.SemaphoreType.DMA,
      ],
  )
  def k(tabL_hbm, tabR_hbm, src_hbm, dst_hbm, zh_hbm, sL_hbm, sR_hbm,
        gidx, didx, rows0, rows1, acc, sem0, sem1, semd0, semd1):
    cid = lax.axis_index("c")
    sid = lax.axis_index("s")
    r0 = sid * STRIPE
    pltpu.sync_copy(src_hbm.at[sid], gidx)
    pltpu.sync_copy(zh_hbm.at[pl.ds(r0, STRIPE)], acc.at[pl.ds(r0, STRIPE)])
    plsc.subcore_barrier()

    def run(tab_hbm):
      pltpu.async_copy(tab_hbm.at[gidx.at[0]], rows0, sem0)
      pltpu.async_copy(tab_hbm.at[gidx.at[1]], rows1, sem1)
      pltpu.async_copy(dst_hbm.at[sid, 0], didx.at[0], semd0)
      pltpu.async_copy(dst_hbm.at[sid, 1], didx.at[1], semd1)

      def body(s, carry):
        g = 2 * s
        pltpu.make_async_copy(tab_hbm.at[gidx.at[0]], rows0, sem0).wait()
        pltpu.make_async_copy(dst_hbm.at[sid, 0], didx.at[0], semd0).wait()
        pltpu.sync_copy(rows0, acc.at[didx.at[0]], add=True)
        pltpu.async_copy(tab_hbm.at[gidx.at[g + 2]], rows0, sem0)
        pltpu.async_copy(dst_hbm.at[sid, g + 2], didx.at[0], semd0)
        pltpu.make_async_copy(tab_hbm.at[gidx.at[1]], rows1, sem1).wait()
        pltpu.make_async_copy(dst_hbm.at[sid, 1], didx.at[1], semd1).wait()
        pltpu.sync_copy(rows1, acc.at[didx.at[1]], add=True)
        pltpu.async_copy(tab_hbm.at[gidx.at[g + 3]], rows1, sem1)
        pltpu.async_copy(dst_hbm.at[sid, g + 3], didx.at[1], semd1)
        return carry

      lax.fori_loop(0, NB // 2, body, 0)
      pltpu.make_async_copy(tab_hbm.at[gidx.at[0]], rows0, sem0).wait()
      pltpu.make_async_copy(tab_hbm.at[gidx.at[1]], rows1, sem1).wait()
      pltpu.make_async_copy(dst_hbm.at[sid, 0], didx.at[0], semd0).wait()
      pltpu.make_async_copy(dst_hbm.at[sid, 1], didx.at[1], semd1).wait()

    @pl.when(cid == 0)
    def _():
      run(tabL_hbm)

    @pl.when(cid == 1)
    def _():
      run(tabR_hbm)

    plsc.subcore_barrier()

    @pl.when(cid == 0)
    def _():
      pltpu.sync_copy(acc.at[pl.ds(r0, STRIPE)], sL_hbm.at[pl.ds(r0, STRIPE)])

    @pl.when(cid == 1)
    def _():
      pltpu.sync_copy(acc.at[pl.ds(r0, STRIPE)], sR_hbm.at[pl.ds(r0, STRIPE)])

  return k(tabL, tabR, src3, dst3, zeros_h)


def _tc_pre(x, W, b2):
  def body(x_ref, w_ref, b_ref, oL_ref, oR_ref):
    res = lax.dot_general(
        x_ref[...], w_ref[...], (((1,), (1,)), ((), ())),
        preferred_element_type=jnp.float32) + b_ref[...]
    oL_ref[...] = res[:, :DH]
    oR_ref[...] = res[:, DH:]

  return pl.pallas_call(
      body,
      grid=(N_NODES // MB,),
      in_specs=[
          pl.BlockSpec((MB, D), lambda i: (i, 0)),
          pl.BlockSpec((D, D), lambda i: (0, 0)),
          pl.BlockSpec((1, D), lambda i: (0, 0)),
      ],
      out_specs=[
          pl.BlockSpec((MB, DH), lambda i: (i, 0)),
          pl.BlockSpec((MB, DH), lambda i: (i, 0)),
      ],
      out_shape=[
          jax.ShapeDtypeStruct((N_NODES, DH), jnp.float32),
          jax.ShapeDtypeStruct((N_NODES, DH), jnp.float32),
      ],
  )(x, W, b2)


def _tc_right(hL, hR, WrA, WrB, b2):
  def body(hl_ref, hr_ref, wa_ref, wb_ref, b_ref, o_ref):
    acc = lax.dot_general(hl_ref[...], wa_ref[...], (((1,), (1,)), ((), ())),
                          preferred_element_type=jnp.float32)
    acc = acc + lax.dot_general(hr_ref[...], wb_ref[...],
                                (((1,), (1,)), ((), ())),
                                preferred_element_type=jnp.float32)
    o_ref[...] = acc + b_ref[...]

  return pl.pallas_call(
      body,
      grid=(N_NODES // MB,),
      in_specs=[
          pl.BlockSpec((MB, DH), lambda i: (i, 0)),
          pl.BlockSpec((MB, DH), lambda i: (i, 0)),
          pl.BlockSpec((D, DH), lambda i: (0, 0)),
          pl.BlockSpec((D, DH), lambda i: (0, 0)),
          pl.BlockSpec((1, D), lambda i: (0, 0)),
      ],
      out_specs=pl.BlockSpec((MB, D), lambda i: (i, 0)),
      out_shape=jax.ShapeDtypeStruct((N_NODES, D), jnp.float32),
  )(hL, hR, WrA, WrB, b2)


def _tc_combine(sumL, sumR, cntA, cntB, r, WlA, WlB, relu, split):
  def body(sl_ref, sr_ref, ca_ref, cb_ref, r_ref, wa_ref, wb_ref, *out_refs):
    cnt = ca_ref[:, 0:1] + cb_ref[:, 0:1]
    inv = 1.0 / jnp.maximum(cnt, 1.0)
    acc = lax.dot_general(sl_ref[...] * inv, wa_ref[...],
                          (((1,), (1,)), ((), ())),
                          preferred_element_type=jnp.float32)
    acc = acc + lax.dot_general(sr_ref[...] * inv, wb_ref[...],
                                (((1,), (1,)), ((), ())),
                                preferred_element_type=jnp.float32)
    acc = acc + r_ref[...]
    if relu:
      acc = jnp.maximum(acc, 0.0)
    if split:
      out_refs[0][...] = acc[:, :DH]
      out_refs[1][...] = acc[:, DH:]
    else:
      out_refs[0][...] = acc

  if split:
    out_specs = [pl.BlockSpec((MB, DH), lambda i: (i, 0)),
                 pl.BlockSpec((MB, DH), lambda i: (i, 0))]
    out_shape = [jax.ShapeDtypeStruct((N_NODES, DH), jnp.float32),
                 jax.ShapeDtypeStruct((N_NODES, DH), jnp.float32)]
  else:
    out_specs = pl.BlockSpec((MB, D), lambda i: (i, 0))
    out_shape = jax.ShapeDtypeStruct((N_NODES, D), jnp.float32)

  return pl.pallas_call(
      body,
      grid=(N_NODES // MB,),
      in_specs=[
          pl.BlockSpec((MB, DH), lambda i: (i, 0)),
          pl.BlockSpec((MB, DH), lambda i: (i, 0)),
          pl.BlockSpec((MB, CW), lambda i: (i, 0)),
          pl.BlockSpec((MB, CW), lambda i: (i, 0)),
          pl.BlockSpec((MB, D), lambda i: (i, 0)),
          pl.BlockSpec((D, DH), lambda i: (0, 0)),
          pl.BlockSpec((D, DH), lambda i: (0, 0)),
      ],
      out_specs=out_specs,
      out_shape=out_shape,
  )(sumL, sumR, cntA, cntB, r, WlA, WlB)


def kernel(x, edge_index, W_pre, b_pre, Wl1, Wr1, b1, Wl2, Wr2, b2):
  src = edge_index[0].astype(jnp.int32)
  dst = edge_index[1].astype(jnp.int32)
  npad = EPSP + 2 * B - EPS
  padg = ((jnp.arange(NSUB, dtype=jnp.int32)[:, None] * 499
           + jnp.arange(npad, dtype=jnp.int32)[None, :]) % N_NODES)
  src3 = jnp.concatenate(
      [src.reshape(NSUB, EPS), padg], axis=1).reshape(NSUB, NB + 2, B)
  padd = (N_NODES
          + (jnp.arange(npad, dtype=jnp.int32) % (NP - N_NODES)))
  dst3 = jnp.concatenate(
      [dst.reshape(NSUB, EPS),
       jnp.broadcast_to(padd, (NSUB, npad))], axis=1).reshape(NSUB, NB + 2, B)
  dst3c = dst.reshape(32, NBC, BC)
  zeros_h = jnp.zeros((NP, DH), jnp.float32)
  ones_h = jnp.ones((BC, DH), jnp.float32)
  bp2 = b_pre.reshape(1, D)
  b12 = b1.reshape(1, D)
  b22 = b2.reshape(1, D)

  cntAf, cntBf = _counts(dst3c, zeros_h, ones_h)
  cntA, cntB = cntAf[:, :CW], cntBf[:, :CW]
  h0L, h0R = _tc_pre(x, W_pre, bp2)
  r1 = _tc_right(h0L, h0R, Wr1[:, :DH], Wr1[:, DH:], b12)
  s1L, s1R = _agg(h0L, h0R, src3, dst3, zeros_h)
  h1L, h1R = _tc_combine(s1L, s1R, cntA, cntB, r1,
                         Wl1[:, :DH], Wl1[:, DH:], True, True)
  r2 = _tc_right(h1L, h1R, Wr2[:, :DH], Wr2[:, DH:], b22)
  s2L, s2R = _agg(h1L, h1R, src3, dst3, zeros_h)
  out = _tc_combine(s2L, s2R, cntA, cntB, r2,
                    Wl2[:, :DH], Wl2[:, DH:], False, False)
  return out


# speedup vs baseline: 7.4020x; 1.0478x over previous
import functools
import jax
import jax.numpy as jnp
from jax import lax
from jax.experimental import pallas as pl
from jax.experimental.pallas import tpu as pltpu
from jax.experimental.pallas import tpu_sc as plsc

N_NODES = 10000
N_EDGES = 160000
D = 256
DH = 128
NSUB = 16
NP = 10240
STRIPE = NP // NSUB
EPS = N_EDGES // NSUB
B = 128
EPSP = 10240
NB = EPSP // B
NBC = (NB + 2) // 2
MB = 1000
CW = 16


def _sc_mesh():
  return plsc.VectorSubcoreMesh(core_axis_name="c", subcore_axis_name="s")


def _counts(dst3, zeros_h, ones_h):
  @functools.partial(
      pl.kernel,
      mesh=_sc_mesh(),
      out_type=[
          jax.ShapeDtypeStruct((NP, DH), jnp.float32),
          jax.ShapeDtypeStruct((NP, DH), jnp.float32),
      ],
      scratch_types=[
          pltpu.VMEM((NB + 2, B), jnp.int32),
          pltpu.VMEM((B, DH), jnp.float32),
          pltpu.VMEM_SHARED((NP, DH), jnp.float32),
      ],
  )
  def k(dst_hbm, zh_hbm, ones_hbm, cntA_hbm, cntB_hbm, didx, ones_v, cacc):
    cid = lax.axis_index("c")
    sid = lax.axis_index("s")
    r0 = sid * STRIPE
    pltpu.sync_copy(zh_hbm.at[pl.ds(r0, STRIPE)], cacc.at[pl.ds(r0, STRIPE)])
    pltpu.sync_copy(ones_hbm, ones_v)
    pltpu.sync_copy(dst_hbm.at[sid], didx)
    plsc.subcore_barrier()

    def body(g, carry):
      pltpu.sync_copy(ones_v, cacc.at[didx.at[2 * g + cid]], add=True)
      return carry

    lax.fori_loop(0, NBC, body, 0)
    plsc.subcore_barrier()

    @pl.when(cid == 0)
    def _():
      pltpu.sync_copy(cacc.at[pl.ds(r0, STRIPE)],
                      cntA_hbm.at[pl.ds(r0, STRIPE)])

    @pl.when(cid == 1)
    def _():
      pltpu.sync_copy(cacc.at[pl.ds(r0, STRIPE)],
                      cntB_hbm.at[pl.ds(r0, STRIPE)])

  return k(dst3, zeros_h, ones_h)


def _agg(tabL, tabR, src3, dst3, zeros_h, tok):
  @functools.partial(
      pl.kernel,
      mesh=_sc_mesh(),
      out_type=[
          jax.ShapeDtypeStruct((NP, DH), jnp.float32),
          jax.ShapeDtypeStruct((NP, DH), jnp.float32),
      ],
      scratch_types=[
          pltpu.VMEM((NB + 2, B), jnp.int32),
          pltpu.VMEM((2, B), jnp.int32),
          pltpu.VMEM((B, DH), jnp.float32),
          pltpu.VMEM((B, DH), jnp.float32),
          pltpu.VMEM_SHARED((NP, DH), jnp.float32),
          pltpu.SemaphoreType.DMA,
          pltpu.SemaphoreType.DMA,
          pltpu.SemaphoreType.DMA,
          pltpu.SemaphoreType.DMA,
      ],
  )
  def k(tabL_hbm, tabR_hbm, src_hbm, dst_hbm, zh_hbm, tok_hbm, sL_hbm, sR_hbm,
        gidx, didx, rows0, rows1, acc, sem0, sem1, semd0, semd1):
    cid = lax.axis_index("c")
    sid = lax.axis_index("s")
    r0 = sid * STRIPE
    pltpu.sync_copy(src_hbm.at[sid], gidx)

    def start(tab_hbm):
      pltpu.async_copy(tab_hbm.at[gidx.at[0]], rows0, sem0)
      pltpu.async_copy(tab_hbm.at[gidx.at[1]], rows1, sem1)
      pltpu.async_copy(dst_hbm.at[sid, 0], didx.at[0], semd0)
      pltpu.async_copy(dst_hbm.at[sid, 1], didx.at[1], semd1)

    @pl.when(cid == 0)
    def _():
      start(tabL_hbm)

    @pl.when(cid == 1)
    def _():
      start(tabR_hbm)

    pltpu.sync_copy(zh_hbm.at[pl.ds(r0, STRIPE)], acc.at[pl.ds(r0, STRIPE)])
    plsc.subcore_barrier()

    def run(tab_hbm):
      def body(s, carry):
        g = 2 * s
        pltpu.make_async_copy(tab_hbm.at[gidx.at[0]], rows0, sem0).wait()
        pltpu.make_async_copy(dst_hbm.at[sid, 0], didx.at[0], semd0).wait()
        pltpu.sync_copy(rows0, acc.at[didx.at[0]], add=True)
        pltpu.async_copy(tab_hbm.at[gidx.at[g + 2]], rows0, sem0)
        pltpu.async_copy(dst_hbm.at[sid, g + 2], didx.at[0], semd0)
        pltpu.make_async_copy(tab_hbm.at[gidx.at[1]], rows1, sem1).wait()
        pltpu.make_async_copy(dst_hbm.at[sid, 1], didx.at[1], semd1).wait()
        pltpu.sync_copy(rows1, acc.at[didx.at[1]], add=True)
        pltpu.async_copy(tab_hbm.at[gidx.at[g + 3]], rows1, sem1)
        pltpu.async_copy(dst_hbm.at[sid, g + 3], didx.at[1], semd1)
        return carry

      lax.fori_loop(0, NB // 2, body, 0)
      pltpu.make_async_copy(tab_hbm.at[gidx.at[0]], rows0, sem0).wait()
      pltpu.make_async_copy(tab_hbm.at[gidx.at[1]], rows1, sem1).wait()
      pltpu.make_async_copy(dst_hbm.at[sid, 0], didx.at[0], semd0).wait()
      pltpu.make_async_copy(dst_hbm.at[sid, 1], didx.at[1], semd1).wait()

    @pl.when(cid == 0)
    def _():
      run(tabL_hbm)

    @pl.when(cid == 1)
    def _():
      run(tabR_hbm)

    plsc.subcore_barrier()

    @pl.when(cid == 0)
    def _():
      pltpu.sync_copy(acc.at[pl.ds(r0, STRIPE)], sL_hbm.at[pl.ds(r0, STRIPE)])

    @pl.when(cid == 1)
    def _():
      pltpu.sync_copy(acc.at[pl.ds(r0, STRIPE)], sR_hbm.at[pl.ds(r0, STRIPE)])

  return k(tabL, tabR, src3, dst3, zeros_h, tok)


def _tc_pre(x, W, b2):
  def body(x_ref, w_ref, b_ref, oL_ref, oR_ref):
    res = lax.dot_general(
        x_ref[...], w_ref[...], (((1,), (1,)), ((), ())),
        preferred_element_type=jnp.float32) + b_ref[...]
    oL_ref[...] = res[:, :DH]
    oR_ref[...] = res[:, DH:]

  return pl.pallas_call(
      body,
      grid=(N_NODES // MB,),
      in_specs=[
          pl.BlockSpec((MB, D), lambda i: (i, 0)),
          pl.BlockSpec((D, D), lambda i: (0, 0)),
          pl.BlockSpec((1, D), lambda i: (0, 0)),
      ],
      out_specs=[
          pl.BlockSpec((MB, DH), lambda i: (i, 0)),
          pl.BlockSpec((MB, DH), lambda i: (i, 0)),
      ],
      out_shape=[
          jax.ShapeDtypeStruct((N_NODES, DH), jnp.float32),
          jax.ShapeDtypeStruct((N_NODES, DH), jnp.float32),
      ],
  )(x, W, b2)


def _tc_right(hL, hR, WrA, WrB, b2):
  def body(hl_ref, hr_ref, wa_ref, wb_ref, b_ref, o_ref):
    acc = lax.dot_general(hl_ref[...], wa_ref[...], (((1,), (1,)), ((), ())),
                          preferred_element_type=jnp.float32)
    acc = acc + lax.dot_general(hr_ref[...], wb_ref[...],
                                (((1,), (1,)), ((), ())),
                                preferred_element_type=jnp.float32)
    o_ref[...] = acc + b_ref[...]

  return pl.pallas_call(
      body,
      grid=(N_NODES // MB,),
      in_specs=[
          pl.BlockSpec((MB, DH), lambda i: (i, 0)),
          pl.BlockSpec((MB, DH), lambda i: (i, 0)),
          pl.BlockSpec((D, DH), lambda i: (0, 0)),
          pl.BlockSpec((D, DH), lambda i: (0, 0)),
          pl.BlockSpec((1, D), lambda i: (0, 0)),
      ],
      out_specs=pl.BlockSpec((MB, D), lambda i: (i, 0)),
      out_shape=jax.ShapeDtypeStruct((N_NODES, D), jnp.float32),
  )(hL, hR, WrA, WrB, b2)


def _tc_combine(sumL, sumR, cntA, cntB, r, WlA, WlB, relu, split):
  def body(sl_ref, sr_ref, ca_ref, cb_ref, r_ref, wa_ref, wb_ref, *out_refs):
    cnt = ca_ref[:, 0:1] + cb_ref[:, 0:1]
    inv = 1.0 / jnp.maximum(cnt, 1.0)
    acc = lax.dot_general(sl_ref[...] * inv, wa_ref[...],
                          (((1,), (1,)), ((), ())),
                          preferred_element_type=jnp.float32)
    acc = acc + lax.dot_general(sr_ref[...] * inv, wb_ref[...],
                                (((1,), (1,)), ((), ())),
                                preferred_element_type=jnp.float32)
    acc = acc + r_ref[...]
    if relu:
      acc = jnp.maximum(acc, 0.0)
    if split:
      out_refs[0][...] = acc[:, :DH]
      out_refs[1][...] = acc[:, DH:]
    else:
      out_refs[0][...] = acc

  if split:
    out_specs = [pl.BlockSpec((MB, DH), lambda i: (i, 0)),
                 pl.BlockSpec((MB, DH), lambda i: (i, 0))]
    out_shape = [jax.ShapeDtypeStruct((N_NODES, DH), jnp.float32),
                 jax.ShapeDtypeStruct((N_NODES, DH), jnp.float32)]
  else:
    out_specs = pl.BlockSpec((MB, D), lambda i: (i, 0))
    out_shape = jax.ShapeDtypeStruct((N_NODES, D), jnp.float32)

  return pl.pallas_call(
      body,
      grid=(N_NODES // MB,),
      in_specs=[
          pl.BlockSpec((MB, DH), lambda i: (i, 0)),
          pl.BlockSpec((MB, DH), lambda i: (i, 0)),
          pl.BlockSpec((MB, CW), lambda i: (i, 0)),
          pl.BlockSpec((MB, CW), lambda i: (i, 0)),
          pl.BlockSpec((MB, D), lambda i: (i, 0)),
          pl.BlockSpec((D, DH), lambda i: (0, 0)),
          pl.BlockSpec((D, DH), lambda i: (0, 0)),
      ],
      out_specs=out_specs,
      out_shape=out_shape,
  )(sumL, sumR, cntA, cntB, r, WlA, WlB)


def kernel(x, edge_index, W_pre, b_pre, Wl1, Wr1, b1, Wl2, Wr2, b2):
  src = edge_index[0].astype(jnp.int32)
  dst = edge_index[1].astype(jnp.int32)
  npad = EPSP + 2 * B - EPS
  padg = ((jnp.arange(NSUB, dtype=jnp.int32)[:, None] * 499
           + jnp.arange(npad, dtype=jnp.int32)[None, :]) % N_NODES)
  src3 = jnp.concatenate(
      [src.reshape(NSUB, EPS), padg], axis=1).reshape(NSUB, NB + 2, B)
  padd = (N_NODES
          + (jnp.arange(npad, dtype=jnp.int32) % (NP - N_NODES)))
  dst3 = jnp.concatenate(
      [dst.reshape(NSUB, EPS),
       jnp.broadcast_to(padd, (NSUB, npad))], axis=1).reshape(NSUB, NB + 2, B)
  zeros_h = jnp.zeros((NP, DH), jnp.float32)
  ones_h = jnp.ones((B, DH), jnp.float32)
  bp2 = b_pre.reshape(1, D)
  b12 = b1.reshape(1, D)
  b22 = b2.reshape(1, D)

  cntAf, cntBf = _counts(dst3, zeros_h, ones_h)
  cntA, cntB = cntAf[:, :CW], cntBf[:, :CW]
  tok = cntAf[:8, :8]
  h0L, h0R = _tc_pre(x, W_pre, bp2)
  r1 = _tc_right(h0L, h0R, Wr1[:, :DH], Wr1[:, DH:], b12)
  s1L, s1R = _agg(h0L, h0R, src3, dst3, zeros_h, tok)
  h1L, h1R = _tc_combine(s1L, s1R, cntA, cntB, r1,
                         Wl1[:, :DH], Wl1[:, DH:], True, True)
  r2 = _tc_right(h1L, h1R, Wr2[:, :DH], Wr2[:, DH:], b22)
  s2L, s2R = _agg(h1L, h1R, src3, dst3, zeros_h, tok)
  out = _tc_combine(s2L, s2R, cntA, cntB, r2,
                    Wl2[:, :DH], Wl2[:, DH:], False, False)
  return out


# speedup vs baseline: 7.4165x; 1.0020x over previous
import functools
import jax
import jax.numpy as jnp
from jax import lax
from jax.experimental import pallas as pl
from jax.experimental.pallas import tpu as pltpu
from jax.experimental.pallas import tpu_sc as plsc

N_NODES = 10000
N_EDGES = 160000
D = 256
DH = 128
NSUB = 16
NP = 10240
STRIPE = NP // NSUB
EPS = N_EDGES // NSUB
B = 128
EPSP = 10240
NB = EPSP // B
NBC = (NB + 2) // 2
MB = 2000
CW = 16


def _sc_mesh():
  return plsc.VectorSubcoreMesh(core_axis_name="c", subcore_axis_name="s")


def _counts(dst3, zeros_h, ones_h):
  @functools.partial(
      pl.kernel,
      mesh=_sc_mesh(),
      out_type=[
          jax.ShapeDtypeStruct((NP, DH), jnp.float32),
          jax.ShapeDtypeStruct((NP, DH), jnp.float32),
      ],
      scratch_types=[
          pltpu.VMEM((NB + 2, B), jnp.int32),
          pltpu.VMEM((B, DH), jnp.float32),
          pltpu.VMEM_SHARED((NP, DH), jnp.float32),
      ],
  )
  def k(dst_hbm, zh_hbm, ones_hbm, cntA_hbm, cntB_hbm, didx, ones_v, cacc):
    cid = lax.axis_index("c")
    sid = lax.axis_index("s")
    r0 = sid * STRIPE
    pltpu.sync_copy(zh_hbm.at[pl.ds(r0, STRIPE)], cacc.at[pl.ds(r0, STRIPE)])
    pltpu.sync_copy(ones_hbm, ones_v)
    pltpu.sync_copy(dst_hbm.at[sid], didx)
    plsc.subcore_barrier()

    def body(g, carry):
      pltpu.sync_copy(ones_v, cacc.at[didx.at[2 * g + cid]], add=True)
      return carry

    lax.fori_loop(0, NBC, body, 0)
    plsc.subcore_barrier()

    @pl.when(cid == 0)
    def _():
      pltpu.sync_copy(cacc.at[pl.ds(r0, STRIPE)],
                      cntA_hbm.at[pl.ds(r0, STRIPE)])

    @pl.when(cid == 1)
    def _():
      pltpu.sync_copy(cacc.at[pl.ds(r0, STRIPE)],
                      cntB_hbm.at[pl.ds(r0, STRIPE)])

  return k(dst3, zeros_h, ones_h)


def _agg(tabL, tabR, src3, dst3, zeros_h, tok):
  @functools.partial(
      pl.kernel,
      mesh=_sc_mesh(),
      out_type=[
          jax.ShapeDtypeStruct((NP, DH), jnp.float32),
          jax.ShapeDtypeStruct((NP, DH), jnp.float32),
      ],
      scratch_types=[
          pltpu.VMEM((NB + 2, B), jnp.int32),
          pltpu.VMEM((2, B), jnp.int32),
          pltpu.VMEM((B, DH), jnp.float32),
          pltpu.VMEM((B, DH), jnp.float32),
          pltpu.VMEM_SHARED((NP, DH), jnp.float32),
          pltpu.SemaphoreType.DMA,
          pltpu.SemaphoreType.DMA,
          pltpu.SemaphoreType.DMA,
          pltpu.SemaphoreType.DMA,
      ],
  )
  def k(tabL_hbm, tabR_hbm, src_hbm, dst_hbm, zh_hbm, tok_hbm, sL_hbm, sR_hbm,
        gidx, didx, rows0, rows1, acc, sem0, sem1, semd0, semd1):
    cid = lax.axis_index("c")
    sid = lax.axis_index("s")
    r0 = sid * STRIPE
    pltpu.sync_copy(src_hbm.at[sid], gidx)

    def start(tab_hbm):
      pltpu.async_copy(tab_hbm.at[gidx.at[0]], rows0, sem0)
      pltpu.async_copy(tab_hbm.at[gidx.at[1]], rows1, sem1)
      pltpu.async_copy(dst_hbm.at[sid, 0], didx.at[0], semd0)
      pltpu.async_copy(dst_hbm.at[sid, 1], didx.at[1], semd1)

    @pl.when(cid == 0)
    def _():
      start(tabL_hbm)

    @pl.when(cid == 1)
    def _():
      start(tabR_hbm)

    pltpu.sync_copy(zh_hbm.at[pl.ds(r0, STRIPE)], acc.at[pl.ds(r0, STRIPE)])
    plsc.subcore_barrier()

    def run(tab_hbm):
      def body(s, carry):
        g = 2 * s
        pltpu.make_async_copy(tab_hbm.at[gidx.at[0]], rows0, sem0).wait()
        pltpu.make_async_copy(dst_hbm.at[sid, 0], didx.at[0], semd0).wait()
        pltpu.sync_copy(rows0, acc.at[didx.at[0]], add=True)
        pltpu.async_copy(tab_hbm.at[gidx.at[g + 2]], rows0, sem0)
        pltpu.async_copy(dst_hbm.at[sid, g + 2], didx.at[0], semd0)
        pltpu.make_async_copy(tab_hbm.at[gidx.at[1]], rows1, sem1).wait()
        pltpu.make_async_copy(dst_hbm.at[sid, 1], didx.at[1], semd1).wait()
        pltpu.sync_copy(rows1, acc.at[didx.at[1]], add=True)
        pltpu.async_copy(tab_hbm.at[gidx.at[g + 3]], rows1, sem1)
        pltpu.async_copy(dst_hbm.at[sid, g + 3], didx.at[1], semd1)
        return carry

      lax.fori_loop(0, NB // 2, body, 0)
      pltpu.make_async_copy(tab_hbm.at[gidx.at[0]], rows0, sem0).wait()
      pltpu.make_async_copy(tab_hbm.at[gidx.at[1]], rows1, sem1).wait()
      pltpu.make_async_copy(dst_hbm.at[sid, 0], didx.at[0], semd0).wait()
      pltpu.make_async_copy(dst_hbm.at[sid, 1], didx.at[1], semd1).wait()

    @pl.when(cid == 0)
    def _():
      run(tabL_hbm)

    @pl.when(cid == 1)
    def _():
      run(tabR_hbm)

    plsc.subcore_barrier()

    @pl.when(cid == 0)
    def _():
      pltpu.sync_copy(acc.at[pl.ds(r0, STRIPE)], sL_hbm.at[pl.ds(r0, STRIPE)])

    @pl.when(cid == 1)
    def _():
      pltpu.sync_copy(acc.at[pl.ds(r0, STRIPE)], sR_hbm.at[pl.ds(r0, STRIPE)])

  return k(tabL, tabR, src3, dst3, zeros_h, tok)


def _tc_pre(x, W, b2):
  def body(x_ref, w_ref, b_ref, oL_ref, oR_ref):
    res = lax.dot_general(
        x_ref[...], w_ref[...], (((1,), (1,)), ((), ())),
        preferred_element_type=jnp.float32) + b_ref[...]
    oL_ref[...] = res[:, :DH]
    oR_ref[...] = res[:, DH:]

  return pl.pallas_call(
      body,
      grid=(N_NODES // MB,),
      in_specs=[
          pl.BlockSpec((MB, D), lambda i: (i, 0)),
          pl.BlockSpec((D, D), lambda i: (0, 0)),
          pl.BlockSpec((1, D), lambda i: (0, 0)),
      ],
      out_specs=[
          pl.BlockSpec((MB, DH), lambda i: (i, 0)),
          pl.BlockSpec((MB, DH), lambda i: (i, 0)),
      ],
      out_shape=[
          jax.ShapeDtypeStruct((N_NODES, DH), jnp.float32),
          jax.ShapeDtypeStruct((N_NODES, DH), jnp.float32),
      ],
  )(x, W, b2)


def _tc_right(hL, hR, WrA, WrB, b2):
  def body(hl_ref, hr_ref, wa_ref, wb_ref, b_ref, o_ref):
    acc = lax.dot_general(hl_ref[...], wa_ref[...], (((1,), (1,)), ((), ())),
                          preferred_element_type=jnp.float32)
    acc = acc + lax.dot_general(hr_ref[...], wb_ref[...],
                                (((1,), (1,)), ((), ())),
                                preferred_element_type=jnp.float32)
    o_ref[...] = acc + b_ref[...]

  return pl.pallas_call(
      body,
      grid=(N_NODES // MB,),
      in_specs=[
          pl.BlockSpec((MB, DH), lambda i: (i, 0)),
          pl.BlockSpec((MB, DH), lambda i: (i, 0)),
          pl.BlockSpec((D, DH), lambda i: (0, 0)),
          pl.BlockSpec((D, DH), lambda i: (0, 0)),
          pl.BlockSpec((1, D), lambda i: (0, 0)),
      ],
      out_specs=pl.BlockSpec((MB, D), lambda i: (i, 0)),
      out_shape=jax.ShapeDtypeStruct((N_NODES, D), jnp.float32),
  )(hL, hR, WrA, WrB, b2)


def _tc_combine(sumL, sumR, cntA, cntB, r, WlA, WlB, relu, split):
  def body(sl_ref, sr_ref, ca_ref, cb_ref, r_ref, wa_ref, wb_ref, *out_refs):
    cnt = ca_ref[:, 0:1] + cb_ref[:, 0:1]
    inv = 1.0 / jnp.maximum(cnt, 1.0)
    acc = lax.dot_general(sl_ref[...] * inv, wa_ref[...],
                          (((1,), (1,)), ((), ())),
                          preferred_element_type=jnp.float32)
    acc = acc + lax.dot_general(sr_ref[...] * inv, wb_ref[...],
                                (((1,), (1,)), ((), ())),
                                preferred_element_type=jnp.float32)
    acc = acc + r_ref[...]
    if relu:
      acc = jnp.maximum(acc, 0.0)
    if split:
      out_refs[0][...] = acc[:, :DH]
      out_refs[1][...] = acc[:, DH:]
    else:
      out_refs[0][...] = acc

  if split:
    out_specs = [pl.BlockSpec((MB, DH), lambda i: (i, 0)),
                 pl.BlockSpec((MB, DH), lambda i: (i, 0))]
    out_shape = [jax.ShapeDtypeStruct((N_NODES, DH), jnp.float32),
                 jax.ShapeDtypeStruct((N_NODES, DH), jnp.float32)]
  else:
    out_specs = pl.BlockSpec((MB, D), lambda i: (i, 0))
    out_shape = jax.ShapeDtypeStruct((N_NODES, D), jnp.float32)

  return pl.pallas_call(
      body,
      grid=(N_NODES // MB,),
      in_specs=[
          pl.BlockSpec((MB, DH), lambda i: (i, 0)),
          pl.BlockSpec((MB, DH), lambda i: (i, 0)),
          pl.BlockSpec((MB, CW), lambda i: (i, 0)),
          pl.BlockSpec((MB, CW), lambda i: (i, 0)),
          pl.BlockSpec((MB, D), lambda i: (i, 0)),
          pl.BlockSpec((D, DH), lambda i: (0, 0)),
          pl.BlockSpec((D, DH), lambda i: (0, 0)),
      ],
      out_specs=out_specs,
      out_shape=out_shape,
  )(sumL, sumR, cntA, cntB, r, WlA, WlB)


def kernel(x, edge_index, W_pre, b_pre, Wl1, Wr1, b1, Wl2, Wr2, b2):
  src = edge_index[0].astype(jnp.int32)
  dst = edge_index[1].astype(jnp.int32)
  npad = EPSP + 2 * B - EPS
  padg = ((jnp.arange(NSUB, dtype=jnp.int32)[:, None] * 499
           + jnp.arange(npad, dtype=jnp.int32)[None, :]) % N_NODES)
  src3 = jnp.concatenate(
      [src.reshape(NSUB, EPS), padg], axis=1).reshape(NSUB, NB + 2, B)
  padd = (N_NODES
          + (jnp.arange(npad, dtype=jnp.int32) % (NP - N_NODES)))
  dst3 = jnp.concatenate(
      [dst.reshape(NSUB, EPS),
       jnp.broadcast_to(padd, (NSUB, npad))], axis=1).reshape(NSUB, NB + 2, B)
  zeros_h = jnp.zeros((NP, DH), jnp.float32)
  ones_h = jnp.ones((B, DH), jnp.float32)
  bp2 = b_pre.reshape(1, D)
  b12 = b1.reshape(1, D)
  b22 = b2.reshape(1, D)

  cntAf, cntBf = _counts(dst3, zeros_h, ones_h)
  cntA, cntB = cntAf[:, :CW], cntBf[:, :CW]
  tok = cntAf[:8, :8]
  h0L, h0R = _tc_pre(x, W_pre, bp2)
  r1 = _tc_right(h0L, h0R, Wr1[:, :DH], Wr1[:, DH:], b12)
  s1L, s1R = _agg(h0L, h0R, src3, dst3, zeros_h, tok)
  h1L, h1R = _tc_combine(s1L, s1R, cntA, cntB, r1,
                         Wl1[:, :DH], Wl1[:, DH:], True, True)
  r2 = _tc_right(h1L, h1R, Wr2[:, :DH], Wr2[:, DH:], b22)
  s2L, s2R = _agg(h1L, h1R, src3, dst3, zeros_h, tok)
  out = _tc_combine(s2L, s2R, cntA, cntB, r2,
                    Wl2[:, :DH], Wl2[:, DH:], False, False)
  return out
